# baseline (speedup 1.0000x reference)
import jax
import jax.numpy as jnp
from jax import lax
from jax.experimental import pallas as pl
from jax.experimental.pallas import tpu as pltpu
from jax.experimental.pallas import tpu_sc as plsc

N = 10000
E = 320000
D = 128
DE = 16
DG = 64
G = 64
H = 128

NC = 2
NS = 16
NW = NC * NS
EPT = E // NW
CH = 80
NCHUNK = EPT // CH
NSB = CH // 16
NP = 10240
RPT = NP // NS
WCR = NP // 64


def _proj_a_body(x_ref, w_ref, o_ref):
    o_ref[...] = jnp.dot(x_ref[...], w_ref[...],
                         preferred_element_type=jnp.float32)


def _proj_b_body(ea_ref, w_ref, b_ref, o_ref):
    o_ref[...] = jnp.dot(ea_ref[...], w_ref[...],
                         preferred_element_type=jnp.float32) + b_ref[...]


def _proj_a(x, w1a):
    blk = 2000
    return pl.pallas_call(
        _proj_a_body,
        grid=(N // blk,),
        in_specs=[
            pl.BlockSpec((blk, D), lambda i: (i, 0)),
            pl.BlockSpec((D, H), lambda i: (0, 0)),
        ],
        out_specs=pl.BlockSpec((blk, H), lambda i: (i, 0)),
        out_shape=jax.ShapeDtypeStruct((N, H), jnp.float32),
    )(x, w1a)


def _proj_b(ea, w1b, b1):
    blk = 4000
    return pl.pallas_call(
        _proj_b_body,
        grid=(E // blk,),
        in_specs=[
            pl.BlockSpec((blk, DE), lambda i: (i, 0)),
            pl.BlockSpec((DE, H), lambda i: (0, 0)),
            pl.BlockSpec((1, H), lambda i: (0, 0)),
        ],
        out_specs=pl.BlockSpec((blk, H), lambda i: (i, 0)),
        out_shape=jax.ShapeDtypeStruct((E, H), jnp.float32),
    )(ea, w1b, b1.reshape(1, H))


def _edge_body(meta_hbm, wts_hbm, a_hbm, b_hbm, zeros_hbm, out_hbm, outwc_hbm,
               s_acc, wc_acc, meta_v, wts_v, a_v, b_v, o_v, o2_v,
               sem_meta, sem_ab, sem_st):
    c = lax.axis_index("c")
    s = lax.axis_index("s")
    wid = s * NC + c
    pltpu.sync_copy(zeros_hbm.at[pl.ds(s * RPT, RPT)],
                    s_acc.at[pl.ds(s * RPT, RPT)])

    @pl.when(s < 5)
    def _zero_wc():
        pltpu.sync_copy(zeros_hbm.at[pl.ds(s * 32, 32)],
                        wc_acc.at[pl.ds(s * 32, 32)])

    plsc.subcore_barrier()

    iota16 = lax.broadcasted_iota(jnp.int32, (16,), 0)

    pltpu.async_copy(meta_hbm.at[wid, 0], meta_v.at[0], sem_meta)
    pltpu.async_copy(wts_hbm.at[wid, 0], wts_v.at[0], sem_meta)

    def chunk(i, carry):
        slot = lax.rem(i, 2)
        nslot = lax.rem(i + 1, 2)
        pltpu.make_async_copy(meta_hbm.at[wid, i], meta_v.at[slot],
                              sem_meta).wait()
        pltpu.make_async_copy(wts_hbm.at[wid, i], wts_v.at[slot],
                              sem_meta).wait()
        pltpu.async_copy(meta_hbm.at[wid, i + 1], meta_v.at[nslot], sem_meta)
        pltpu.async_copy(wts_hbm.at[wid, i + 1], wts_v.at[nslot], sem_meta)
        ga = pltpu.async_copy(a_hbm.at[meta_v.at[slot, 0]], a_v, sem_ab)
        gb = pltpu.async_copy(b_hbm.at[wid, i], b_v, sem_ab)
        ga.wait()
        gb.wait()

        @pl.when(i > 0)
        def _drain():
            pltpu.make_async_copy(o_v, s_acc.at[meta_v.at[slot, 1]],
                                  sem_st).wait()
            pltpu.make_async_copy(o2_v, wc_acc.at[meta_v.at[slot, 2]],
                                  sem_st).wait()

        def subblock(sb, carry2):
            wv = wts_v[slot, pl.ds(sb * 16, 16)]
            cv = meta_v[slot, 1, pl.ds(sb * 16, 16)]
            l2 = 2 * (cv % 64)
            for ee in range(16):
                e = sb * 16 + ee
                w = wv[ee]
                t = l2[ee]
                for j in range(D // 16):
                    lo = iota16 + (16 * j)
                    o2_v[e, pl.ds(16 * j, 16)] = jnp.where(
                        lo == t, w, jnp.where(lo == t + 1, 1.0, 0.0))
                for j in range(D // 16):
                    sl = pl.ds(16 * j, 16)
                    o_v[e, sl] = jnp.maximum(a_v[e, sl] + b_v[e, sl],
                                             0.0) * w
            return carry2

        lax.fori_loop(0, NSB, subblock, 0, unroll=True)
        pltpu.async_copy(o_v, s_acc.at[meta_v.at[slot, 1]], sem_st,
                         add=True)
        pltpu.async_copy(o2_v, wc_acc.at[meta_v.at[slot, 2]], sem_st,
                         add=True)
        return carry

    lax.fori_loop(0, NCHUNK, chunk, 0, unroll=False)
    lslot = lax.rem(NCHUNK - 1, 2)
    pltpu.make_async_copy(o_v, s_acc.at[meta_v.at[lslot, 1]], sem_st).wait()
    pltpu.make_async_copy(o2_v, wc_acc.at[meta_v.at[lslot, 2]],
                          sem_st).wait()
    pltpu.make_async_copy(meta_hbm.at[wid, NCHUNK],
                          meta_v.at[lax.rem(NCHUNK, 2)], sem_meta).wait()
    pltpu.make_async_copy(wts_hbm.at[wid, NCHUNK],
                          wts_v.at[lax.rem(NCHUNK, 2)], sem_meta).wait()
    plsc.subcore_barrier()
    pltpu.sync_copy(s_acc.at[pl.ds(s * RPT, RPT)],
                    out_hbm.at[c, pl.ds(s * RPT, RPT)])

    @pl.when(s == 0)
    def _dump_wc():
        pltpu.sync_copy(wc_acc, outwc_hbm.at[c])


def _edge_stage(meta, wts, a, b, zeros):
    mesh = plsc.VectorSubcoreMesh(core_axis_name="c", subcore_axis_name="s",
                                  num_cores=NC, num_subcores=NS)
    fn = pl.kernel(
        _edge_body,
        out_type=(jax.ShapeDtypeStruct((NC, NP, D), jnp.float32),
                  jax.ShapeDtypeStruct((NC, WCR, D), jnp.float32)),
        mesh=mesh,
        scratch_types=[
            pltpu.VMEM_SHARED((NP, D), jnp.float32),
            pltpu.VMEM_SHARED((WCR, D), jnp.float32),
            pltpu.VMEM((2, 3, CH), jnp.int32),
            pltpu.VMEM((2, CH), jnp.float32),
            pltpu.VMEM((CH, D), jnp.float32),
            pltpu.VMEM((CH, D), jnp.float32),
            pltpu.VMEM((CH, D), jnp.float32),
            pltpu.VMEM((CH, D), jnp.float32),
            pltpu.SemaphoreType.DMA,
            pltpu.SemaphoreType.DMA,
            pltpu.SemaphoreType.DMA,
        ],
        compiler_params=pltpu.CompilerParams(use_tc_tiling_on_sc=False),
    )
    return fn(meta, wts, a, b, zeros)


def _update_body(x_ref, s0_ref, s1_ref, wc0_ref, wc1_ref,
                 nb_ref, u_ref,
                 w2_ref, b2_ref, w3a_ref, w3b_ref, w3c_ref, b3_ref,
                 w4_ref, b4_ref, o_ref):
    blk = x_ref.shape[0]
    wrows = blk // 64
    msg = s0_ref[...] + s1_ref[...]
    wcp = wc0_ref[...] + wc1_ref[...]
    rep_oh = (lax.broadcasted_iota(jnp.int32, (blk, wrows), 0) // 64
              == lax.broadcasted_iota(jnp.int32, (blk, wrows), 1)
              ).astype(jnp.float32)
    rep = jnp.dot(rep_oh, wcp, preferred_element_type=jnp.float32)
    lane2 = 2 * (lax.broadcasted_iota(jnp.int32, (blk, D), 0) % 64)
    lid = lax.broadcasted_iota(jnp.int32, (blk, D), 1)
    wsum = jnp.sum(jnp.where(lid == lane2, rep, 0.0), axis=1, keepdims=True)
    cnt = jnp.sum(jnp.where(lid == lane2 + 1, rep, 0.0),
                  axis=1, keepdims=True)
    rc = 1.0 / jnp.maximum(cnt, 1.0)
    recv = (jnp.dot(msg, w2_ref[...], preferred_element_type=jnp.float32)
            + wsum * b2_ref[...]) * rc
    nb = nb_ref[0, 0, :]
    onehot = (nb[:, None] == lax.broadcasted_iota(jnp.int32, (blk, G), 1)
              ).astype(jnp.float32)
    uproj = jnp.dot(u_ref[...], w3c_ref[...],
                    preferred_element_type=jnp.float32)
    pre = (jnp.dot(x_ref[...], w3a_ref[...],
                   preferred_element_type=jnp.float32)
           + jnp.dot(recv, w3b_ref[...], preferred_element_type=jnp.float32)
           + jnp.dot(onehot, uproj, preferred_element_type=jnp.float32)
           + b3_ref[...])
    h2 = jnp.maximum(pre, 0.0)
    o_ref[...] = (jnp.dot(h2, w4_ref[...], preferred_element_type=jnp.float32)
                  + b4_ref[...])


def _update_stage(xp, s_parts, wc_parts, nb3, u,
                  w2, b2, w3a, w3b, w3c, b3, w4, b4):
    blk = 2048
    grid = NP // blk
    wrows = blk // 64
    return pl.pallas_call(
        _update_body,
        grid=(grid,),
        in_specs=[
            pl.BlockSpec((blk, D), lambda i: (i, 0)),
            pl.BlockSpec((blk, D), lambda i: (i, 0)),
            pl.BlockSpec((blk, D), lambda i: (i, 0)),
            pl.BlockSpec((wrows, D), lambda i: (i, 0)),
            pl.BlockSpec((wrows, D), lambda i: (i, 0)),
            pl.BlockSpec((1, 1, blk), lambda i: (i, 0, 0)),
            pl.BlockSpec((G, DG), lambda i: (0, 0)),
            pl.BlockSpec((D, D), lambda i: (0, 0)),
            pl.BlockSpec((1, D), lambda i: (0, 0)),
            pl.BlockSpec((D, D), lambda i: (0, 0)),
            pl.BlockSpec((D, D), lambda i: (0, 0)),
            pl.BlockSpec((DG, D), lambda i: (0, 0)),
            pl.BlockSpec((1, D), lambda i: (0, 0)),
            pl.BlockSpec((D, D), lambda i: (0, 0)),
            pl.BlockSpec((1, D), lambda i: (0, 0)),
        ],
        out_specs=pl.BlockSpec((blk, D), lambda i: (i, 0)),
        out_shape=jax.ShapeDtypeStruct((NP, D), jnp.float32),
    )(xp, s_parts[0], s_parts[1], wc_parts[0], wc_parts[1],
      nb3, u, w2, b2.reshape(1, D),
      w3a, w3b, w3c, b3.reshape(1, D), w4, b4.reshape(1, D))


def kernel(x, edge_index, edge_attr, u, node_batch, wts,
           W1, b1, W2, b2, W3, b3, W4, b4):
    row = edge_index[0].astype(jnp.int32)
    col = edge_index[1].astype(jnp.int32)
    w = wts[:, 0]

    a = _proj_a(x, W1[:D])
    b = _proj_b(edge_attr, W1[D:], b1)

    meta = jnp.stack([row.reshape(NW, NCHUNK, CH),
                      col.reshape(NW, NCHUNK, CH),
                      (col // 64).reshape(NW, NCHUNK, CH)], axis=2)
    meta = jnp.concatenate(
        [meta, jnp.zeros((NW, 1, 3, CH), jnp.int32)], axis=1)
    wpad = jnp.concatenate(
        [w.reshape(NW, NCHUNK, CH), jnp.zeros((NW, 1, CH), jnp.float32)],
        axis=1)
    b4d = b.reshape(NW, NCHUNK, CH, H)
    zeros = jnp.zeros((NP, D), jnp.float32)

    s_parts, wc_parts = _edge_stage(meta, wpad, a, b4d, zeros)

    xp = jnp.concatenate([x, jnp.zeros((NP - N, D), jnp.float32)], axis=0)
    nbp = jnp.concatenate([node_batch,
                           jnp.zeros((NP - N,), jnp.int32)], axis=0)
    nb3 = nbp.reshape(NP // 2048, 1, 2048)
    out = _update_stage(xp, s_parts, wc_parts, nb3, u,
                        W2, b2, W3[:D], W3[D:2 * D], W3[2 * D:], b3, W4, b4)
    return out[:N]


# speedup vs baseline: 1.6833x; 1.6833x over previous
import jax
import jax.numpy as jnp
from jax import lax
from jax.experimental import pallas as pl
from jax.experimental.pallas import tpu as pltpu
from jax.experimental.pallas import tpu_sc as plsc

N = 10000
E = 320000
D = 128
DE = 16
DG = 64
G = 64
H = 128

NC = 2
NS = 16
NW = NC * NS
EPT = E // NW
CH = 80
NCHUNK = EPT // CH
NSB = CH // 16
NP = 10240
RPT = NP // NS
WCR = NP // 64


def _proj_a_body(x_ref, w_ref, o_ref):
    o_ref[...] = jnp.dot(x_ref[...], w_ref[...],
                         preferred_element_type=jnp.float32)


def _proj_b_body(ea_ref, w_ref, b_ref, o_ref):
    o_ref[...] = jnp.dot(ea_ref[...], w_ref[...],
                         preferred_element_type=jnp.float32) + b_ref[...]


def _proj_a(x, w1a):
    blk = 2000
    return pl.pallas_call(
        _proj_a_body,
        grid=(N // blk,),
        in_specs=[
            pl.BlockSpec((blk, D), lambda i: (i, 0)),
            pl.BlockSpec((D, H), lambda i: (0, 0)),
        ],
        out_specs=pl.BlockSpec((blk, H), lambda i: (i, 0)),
        out_shape=jax.ShapeDtypeStruct((N, H), jnp.float32),
    )(x, w1a)


def _proj_b(ea, w1b, b1):
    blk = 4000
    return pl.pallas_call(
        _proj_b_body,
        grid=(E // blk,),
        in_specs=[
            pl.BlockSpec((blk, DE), lambda i: (i, 0)),
            pl.BlockSpec((DE, H), lambda i: (0, 0)),
            pl.BlockSpec((1, H), lambda i: (0, 0)),
        ],
        out_specs=pl.BlockSpec((blk, H), lambda i: (i, 0)),
        out_shape=jax.ShapeDtypeStruct((E, H), jnp.float32),
    )(ea, w1b, b1.reshape(1, H))


def _edge_body(meta_hbm, wts_hbm, a_hbm, b_hbm, zeros_hbm, out_hbm, outwc_hbm,
               s_acc, wc_acc, meta_v, wts_v, a_v, b_v, o_v, o2_v,
               sem_meta, sem_ab, sem_st):
    c = lax.axis_index("c")
    s = lax.axis_index("s")
    wid = s * NC + c
    pltpu.sync_copy(zeros_hbm.at[pl.ds(s * RPT, RPT)],
                    s_acc.at[pl.ds(s * RPT, RPT)])

    @pl.when(s < 5)
    def _zero_wc():
        pltpu.sync_copy(zeros_hbm.at[pl.ds(s * 32, 32)],
                        wc_acc.at[pl.ds(s * 32, 32)])

    plsc.subcore_barrier()

    iota16 = lax.broadcasted_iota(jnp.int32, (16,), 0)

    pltpu.async_copy(meta_hbm.at[wid, 0], meta_v.at[0], sem_meta)
    pltpu.async_copy(wts_hbm.at[wid, 0], wts_v.at[0], sem_meta)

    def chunk(i, carry):
        slot = lax.rem(i, 2)
        nslot = lax.rem(i + 1, 2)
        pltpu.make_async_copy(meta_hbm.at[wid, i], meta_v.at[slot],
                              sem_meta).wait()
        pltpu.make_async_copy(wts_hbm.at[wid, i], wts_v.at[slot],
                              sem_meta).wait()
        pltpu.async_copy(meta_hbm.at[wid, i + 1], meta_v.at[nslot], sem_meta)
        pltpu.async_copy(wts_hbm.at[wid, i + 1], wts_v.at[nslot], sem_meta)
        ga = pltpu.async_copy(a_hbm.at[meta_v.at[slot, 0]], a_v, sem_ab)
        gb = pltpu.async_copy(b_hbm.at[wid, i], b_v, sem_ab)
        ga.wait()
        gb.wait()

        @pl.when(i > 0)
        def _drain():
            pltpu.make_async_copy(o_v, s_acc.at[meta_v.at[slot, 1]],
                                  sem_st).wait()
            pltpu.make_async_copy(o2_v, wc_acc.at[meta_v.at[slot, 2]],
                                  sem_st).wait()

        def subblock(sb, carry2):
            wv = wts_v[slot, pl.ds(sb * 16, 16)]
            cv = meta_v[slot, 1, pl.ds(sb * 16, 16)]
            l2 = 2 * (cv % 64)
            for ee in range(16):
                e = sb * 16 + ee
                w = wv[ee]
                t = l2[ee]
                for j in range(D // 16):
                    lo = iota16 + (16 * j)
                    o2_v[e, pl.ds(16 * j, 16)] = jnp.where(
                        lo == t, w, jnp.where(lo == t + 1, 1.0, 0.0))
                for j in range(D // 16):
                    sl = pl.ds(16 * j, 16)
                    o_v[e, sl] = jnp.maximum(a_v[e, sl] + b_v[e, sl],
                                             0.0) * w
            return carry2

        lax.fori_loop(0, NSB, subblock, 0, unroll=False)
        pltpu.async_copy(o_v, s_acc.at[meta_v.at[slot, 1]], sem_st,
                         add=True)
        pltpu.async_copy(o2_v, wc_acc.at[meta_v.at[slot, 2]], sem_st,
                         add=True)
        return carry

    lax.fori_loop(0, NCHUNK, chunk, 0, unroll=False)
    lslot = lax.rem(NCHUNK - 1, 2)
    pltpu.make_async_copy(o_v, s_acc.at[meta_v.at[lslot, 1]], sem_st).wait()
    pltpu.make_async_copy(o2_v, wc_acc.at[meta_v.at[lslot, 2]],
                          sem_st).wait()
    pltpu.make_async_copy(meta_hbm.at[wid, NCHUNK],
                          meta_v.at[lax.rem(NCHUNK, 2)], sem_meta).wait()
    pltpu.make_async_copy(wts_hbm.at[wid, NCHUNK],
                          wts_v.at[lax.rem(NCHUNK, 2)], sem_meta).wait()
    plsc.subcore_barrier()
    pltpu.sync_copy(s_acc.at[pl.ds(s * RPT, RPT)],
                    out_hbm.at[c, pl.ds(s * RPT, RPT)])

    @pl.when(s == 0)
    def _dump_wc():
        pltpu.sync_copy(wc_acc, outwc_hbm.at[c])


def _edge_stage(meta, wts, a, b, zeros):
    mesh = plsc.VectorSubcoreMesh(core_axis_name="c", subcore_axis_name="s",
                                  num_cores=NC, num_subcores=NS)
    fn = pl.kernel(
        _edge_body,
        out_type=(jax.ShapeDtypeStruct((NC, NP, D), jnp.float32),
                  jax.ShapeDtypeStruct((NC, WCR, D), jnp.float32)),
        mesh=mesh,
        scratch_types=[
            pltpu.VMEM_SHARED((NP, D), jnp.float32),
            pltpu.VMEM_SHARED((WCR, D), jnp.float32),
            pltpu.VMEM((2, 3, CH), jnp.int32),
            pltpu.VMEM((2, CH), jnp.float32),
            pltpu.VMEM((CH, D), jnp.float32),
            pltpu.VMEM((CH, D), jnp.float32),
            pltpu.VMEM((CH, D), jnp.float32),
            pltpu.VMEM((CH, D), jnp.float32),
            pltpu.SemaphoreType.DMA,
            pltpu.SemaphoreType.DMA,
            pltpu.SemaphoreType.DMA,
        ],
        compiler_params=pltpu.CompilerParams(use_tc_tiling_on_sc=False),
    )
    return fn(meta, wts, a, b, zeros)


def _update_body(x_ref, s0_ref, s1_ref, wc0_ref, wc1_ref,
                 nb_ref, u_ref,
                 w2_ref, b2_ref, w3a_ref, w3b_ref, w3c_ref, b3_ref,
                 w4_ref, b4_ref, o_ref):
    blk = x_ref.shape[0]
    wrows = blk // 64
    msg = s0_ref[...] + s1_ref[...]
    wcp = wc0_ref[...] + wc1_ref[...]
    rep_oh = (lax.broadcasted_iota(jnp.int32, (blk, wrows), 0) // 64
              == lax.broadcasted_iota(jnp.int32, (blk, wrows), 1)
              ).astype(jnp.float32)
    rep = jnp.dot(rep_oh, wcp, preferred_element_type=jnp.float32)
    lane2 = 2 * (lax.broadcasted_iota(jnp.int32, (blk, D), 0) % 64)
    lid = lax.broadcasted_iota(jnp.int32, (blk, D), 1)
    wsum = jnp.sum(jnp.where(lid == lane2, rep, 0.0), axis=1, keepdims=True)
    cnt = jnp.sum(jnp.where(lid == lane2 + 1, rep, 0.0),
                  axis=1, keepdims=True)
    rc = 1.0 / jnp.maximum(cnt, 1.0)
    recv = (jnp.dot(msg, w2_ref[...], preferred_element_type=jnp.float32)
            + wsum * b2_ref[...]) * rc
    nb = nb_ref[0, 0, :]
    onehot = (nb[:, None] == lax.broadcasted_iota(jnp.int32, (blk, G), 1)
              ).astype(jnp.float32)
    uproj = jnp.dot(u_ref[...], w3c_ref[...],
                    preferred_element_type=jnp.float32)
    pre = (jnp.dot(x_ref[...], w3a_ref[...],
                   preferred_element_type=jnp.float32)
           + jnp.dot(recv, w3b_ref[...], preferred_element_type=jnp.float32)
           + jnp.dot(onehot, uproj, preferred_element_type=jnp.float32)
           + b3_ref[...])
    h2 = jnp.maximum(pre, 0.0)
    o_ref[...] = (jnp.dot(h2, w4_ref[...], preferred_element_type=jnp.float32)
                  + b4_ref[...])


def _update_stage(xp, s_parts, wc_parts, nb3, u,
                  w2, b2, w3a, w3b, w3c, b3, w4, b4):
    blk = 2048
    grid = NP // blk
    wrows = blk // 64
    return pl.pallas_call(
        _update_body,
        grid=(grid,),
        in_specs=[
            pl.BlockSpec((blk, D), lambda i: (i, 0)),
            pl.BlockSpec((blk, D), lambda i: (i, 0)),
            pl.BlockSpec((blk, D), lambda i: (i, 0)),
            pl.BlockSpec((wrows, D), lambda i: (i, 0)),
            pl.BlockSpec((wrows, D), lambda i: (i, 0)),
            pl.BlockSpec((1, 1, blk), lambda i: (i, 0, 0)),
            pl.BlockSpec((G, DG), lambda i: (0, 0)),
            pl.BlockSpec((D, D), lambda i: (0, 0)),
            pl.BlockSpec((1, D), lambda i: (0, 0)),
            pl.BlockSpec((D, D), lambda i: (0, 0)),
            pl.BlockSpec((D, D), lambda i: (0, 0)),
            pl.BlockSpec((DG, D), lambda i: (0, 0)),
            pl.BlockSpec((1, D), lambda i: (0, 0)),
            pl.BlockSpec((D, D), lambda i: (0, 0)),
            pl.BlockSpec((1, D), lambda i: (0, 0)),
        ],
        out_specs=pl.BlockSpec((blk, D), lambda i: (i, 0)),
        out_shape=jax.ShapeDtypeStruct((NP, D), jnp.float32),
    )(xp, s_parts[0], s_parts[1], wc_parts[0], wc_parts[1],
      nb3, u, w2, b2.reshape(1, D),
      w3a, w3b, w3c, b3.reshape(1, D), w4, b4.reshape(1, D))


def kernel(x, edge_index, edge_attr, u, node_batch, wts,
           W1, b1, W2, b2, W3, b3, W4, b4):
    row = edge_index[0].astype(jnp.int32)
    col = edge_index[1].astype(jnp.int32)
    w = wts[:, 0]

    a = _proj_a(x, W1[:D])
    b = _proj_b(edge_attr, W1[D:], b1)

    meta = jnp.stack([row.reshape(NW, NCHUNK, CH),
                      col.reshape(NW, NCHUNK, CH),
                      (col // 64).reshape(NW, NCHUNK, CH)], axis=2)
    meta = jnp.concatenate(
        [meta, jnp.zeros((NW, 1, 3, CH), jnp.int32)], axis=1)
    wpad = jnp.concatenate(
        [w.reshape(NW, NCHUNK, CH), jnp.zeros((NW, 1, CH), jnp.float32)],
        axis=1)
    b4d = b.reshape(NW, NCHUNK, CH, H)
    zeros = jnp.zeros((NP, D), jnp.float32)

    s_parts, wc_parts = _edge_stage(meta, wpad, a, b4d, zeros)

    xp = jnp.concatenate([x, jnp.zeros((NP - N, D), jnp.float32)], axis=0)
    nbp = jnp.concatenate([node_batch,
                           jnp.zeros((NP - N,), jnp.int32)], axis=0)
    nb3 = nbp.reshape(NP // 2048, 1, 2048)
    out = _update_stage(xp, s_parts, wc_parts, nb3, u,
                        W2, b2, W3[:D], W3[D:2 * D], W3[2 * D:], b3, W4, b4)
    return out[:N]


# speedup vs baseline: 1.6851x; 1.0010x over previous
import jax
import jax.numpy as jnp
from jax import lax
from jax.experimental import pallas as pl
from jax.experimental.pallas import tpu as pltpu
from jax.experimental.pallas import tpu_sc as plsc

N = 10000
E = 320000
D = 128
DE = 16
DG = 64
G = 64
H = 128

NC = 2
NS = 16
NW = NC * NS
EPT = E // NW
CH = 80
NCHUNK = EPT // CH
NSB = CH // 16
NP = 10240
RPT = NP // NS
WCR = NP // 64


def _proj_a_body(x_ref, w_ref, o_ref):
    o_ref[...] = jnp.dot(x_ref[...], w_ref[...],
                         preferred_element_type=jnp.float32)


def _proj_b_body(ea_ref, w_ref, b_ref, o_ref):
    o_ref[...] = jnp.dot(ea_ref[...], w_ref[...],
                         preferred_element_type=jnp.float32) + b_ref[...]


def _proj_a(x, w1a):
    blk = 2000
    return pl.pallas_call(
        _proj_a_body,
        grid=(N // blk,),
        in_specs=[
            pl.BlockSpec((blk, D), lambda i: (i, 0)),
            pl.BlockSpec((D, H), lambda i: (0, 0)),
        ],
        out_specs=pl.BlockSpec((blk, H), lambda i: (i, 0)),
        out_shape=jax.ShapeDtypeStruct((N, H), jnp.float32),
    )(x, w1a)


def _proj_b(ea, w1b, b1):
    blk = 4000
    return pl.pallas_call(
        _proj_b_body,
        grid=(E // blk,),
        in_specs=[
            pl.BlockSpec((blk, DE), lambda i: (i, 0)),
            pl.BlockSpec((DE, H), lambda i: (0, 0)),
            pl.BlockSpec((1, H), lambda i: (0, 0)),
        ],
        out_specs=pl.BlockSpec((blk, H), lambda i: (i, 0)),
        out_shape=jax.ShapeDtypeStruct((E, H), jnp.float32),
    )(ea, w1b, b1.reshape(1, H))


def _edge_body(meta_hbm, wts_hbm, a_hbm, b_hbm, zeros_hbm, out_hbm, outwc_hbm,
               s_acc, wc_acc, meta_v, wts_v, a_v, b_v, o_v, o2_v,
               sem_meta, sem_ab, sem_st):
    c = lax.axis_index("c")
    s = lax.axis_index("s")
    wid = s * NC + c
    pltpu.sync_copy(zeros_hbm.at[pl.ds(s * RPT, RPT)],
                    s_acc.at[pl.ds(s * RPT, RPT)])

    @pl.when(s < 5)
    def _zero_wc():
        pltpu.sync_copy(zeros_hbm.at[pl.ds(s * 32, 32)],
                        wc_acc.at[pl.ds(s * 32, 32)])

    plsc.subcore_barrier()

    iota16 = lax.broadcasted_iota(jnp.int32, (16,), 0)

    pltpu.async_copy(meta_hbm.at[wid, 0], meta_v.at[0], sem_meta)
    pltpu.async_copy(wts_hbm.at[wid, 0], wts_v.at[0], sem_meta)

    def chunk(i, carry):
        slot = lax.rem(i, 2)
        nslot = lax.rem(i + 1, 2)
        pltpu.make_async_copy(meta_hbm.at[wid, i], meta_v.at[slot],
                              sem_meta).wait()
        pltpu.make_async_copy(wts_hbm.at[wid, i], wts_v.at[slot],
                              sem_meta).wait()
        ga = pltpu.async_copy(a_hbm.at[meta_v.at[slot, 0]], a_v, sem_ab)
        gb = pltpu.async_copy(b_hbm.at[wid, i], b_v, sem_ab)

        @pl.when(i > 0)
        def _drain():
            pltpu.make_async_copy(o_v, s_acc.at[meta_v.at[slot, 1]],
                                  sem_st).wait()
            pltpu.make_async_copy(o2_v, wc_acc.at[meta_v.at[slot, 2]],
                                  sem_st).wait()

        pltpu.async_copy(meta_hbm.at[wid, i + 1], meta_v.at[nslot], sem_meta)
        pltpu.async_copy(wts_hbm.at[wid, i + 1], wts_v.at[nslot], sem_meta)
        ga.wait()
        gb.wait()

        def subblock(sb, carry2):
            wv = wts_v[slot, pl.ds(sb * 16, 16)]
            cv = meta_v[slot, 1, pl.ds(sb * 16, 16)]
            l2 = 2 * (cv % 64)
            for ee in range(16):
                e = sb * 16 + ee
                w = wv[ee]
                t = l2[ee]
                for j in range(D // 16):
                    lo = iota16 + (16 * j)
                    o2_v[e, pl.ds(16 * j, 16)] = jnp.where(
                        lo == t, w, jnp.where(lo == t + 1, 1.0, 0.0))
                for j in range(D // 16):
                    sl = pl.ds(16 * j, 16)
                    o_v[e, sl] = jnp.maximum(a_v[e, sl] + b_v[e, sl],
                                             0.0) * w
            return carry2

        lax.fori_loop(0, NSB, subblock, 0, unroll=False)
        pltpu.async_copy(o_v, s_acc.at[meta_v.at[slot, 1]], sem_st,
                         add=True)
        pltpu.async_copy(o2_v, wc_acc.at[meta_v.at[slot, 2]], sem_st,
                         add=True)
        return carry

    lax.fori_loop(0, NCHUNK, chunk, 0, unroll=False)
    lslot = lax.rem(NCHUNK - 1, 2)
    pltpu.make_async_copy(o_v, s_acc.at[meta_v.at[lslot, 1]], sem_st).wait()
    pltpu.make_async_copy(o2_v, wc_acc.at[meta_v.at[lslot, 2]],
                          sem_st).wait()
    pltpu.make_async_copy(meta_hbm.at[wid, NCHUNK],
                          meta_v.at[lax.rem(NCHUNK, 2)], sem_meta).wait()
    pltpu.make_async_copy(wts_hbm.at[wid, NCHUNK],
                          wts_v.at[lax.rem(NCHUNK, 2)], sem_meta).wait()
    plsc.subcore_barrier()
    pltpu.sync_copy(s_acc.at[pl.ds(s * RPT, RPT)],
                    out_hbm.at[c, pl.ds(s * RPT, RPT)])

    @pl.when(s == 0)
    def _dump_wc():
        pltpu.sync_copy(wc_acc, outwc_hbm.at[c])


def _edge_stage(meta, wts, a, b, zeros):
    mesh = plsc.VectorSubcoreMesh(core_axis_name="c", subcore_axis_name="s",
                                  num_cores=NC, num_subcores=NS)
    fn = pl.kernel(
        _edge_body,
        out_type=(jax.ShapeDtypeStruct((NC, NP, D), jnp.float32),
                  jax.ShapeDtypeStruct((NC, WCR, D), jnp.float32)),
        mesh=mesh,
        scratch_types=[
            pltpu.VMEM_SHARED((NP, D), jnp.float32),
            pltpu.VMEM_SHARED((WCR, D), jnp.float32),
            pltpu.VMEM((2, 3, CH), jnp.int32),
            pltpu.VMEM((2, CH), jnp.float32),
            pltpu.VMEM((CH, D), jnp.float32),
            pltpu.VMEM((CH, D), jnp.float32),
            pltpu.VMEM((CH, D), jnp.float32),
            pltpu.VMEM((CH, D), jnp.float32),
            pltpu.SemaphoreType.DMA,
            pltpu.SemaphoreType.DMA,
            pltpu.SemaphoreType.DMA,
        ],
        compiler_params=pltpu.CompilerParams(use_tc_tiling_on_sc=False),
    )
    return fn(meta, wts, a, b, zeros)


def _update_body(x_ref, s0_ref, s1_ref, wc0_ref, wc1_ref,
                 nb_ref, u_ref,
                 w2_ref, b2_ref, w3a_ref, w3b_ref, w3c_ref, b3_ref,
                 w4_ref, b4_ref, o_ref):
    blk = x_ref.shape[0]
    wrows = blk // 64
    msg = s0_ref[...] + s1_ref[...]
    wcp = wc0_ref[...] + wc1_ref[...]
    rep_oh = (lax.broadcasted_iota(jnp.int32, (blk, wrows), 0) // 64
              == lax.broadcasted_iota(jnp.int32, (blk, wrows), 1)
              ).astype(jnp.float32)
    rep = jnp.dot(rep_oh, wcp, preferred_element_type=jnp.float32)
    lane2 = 2 * (lax.broadcasted_iota(jnp.int32, (blk, D), 0) % 64)
    lid = lax.broadcasted_iota(jnp.int32, (blk, D), 1)
    wsum = jnp.sum(jnp.where(lid == lane2, rep, 0.0), axis=1, keepdims=True)
    cnt = jnp.sum(jnp.where(lid == lane2 + 1, rep, 0.0),
                  axis=1, keepdims=True)
    rc = 1.0 / jnp.maximum(cnt, 1.0)
    recv = (jnp.dot(msg, w2_ref[...], preferred_element_type=jnp.float32)
            + wsum * b2_ref[...]) * rc
    nb = nb_ref[0, 0, :]
    onehot = (nb[:, None] == lax.broadcasted_iota(jnp.int32, (blk, G), 1)
              ).astype(jnp.float32)
    uproj = jnp.dot(u_ref[...], w3c_ref[...],
                    preferred_element_type=jnp.float32)
    pre = (jnp.dot(x_ref[...], w3a_ref[...],
                   preferred_element_type=jnp.float32)
           + jnp.dot(recv, w3b_ref[...], preferred_element_type=jnp.float32)
           + jnp.dot(onehot, uproj, preferred_element_type=jnp.float32)
           + b3_ref[...])
    h2 = jnp.maximum(pre, 0.0)
    o_ref[...] = (jnp.dot(h2, w4_ref[...], preferred_element_type=jnp.float32)
                  + b4_ref[...])


def _update_stage(xp, s_parts, wc_parts, nb3, u,
                  w2, b2, w3a, w3b, w3c, b3, w4, b4):
    blk = 2048
    grid = NP // blk
    wrows = blk // 64
    return pl.pallas_call(
        _update_body,
        grid=(grid,),
        in_specs=[
            pl.BlockSpec((blk, D), lambda i: (i, 0)),
            pl.BlockSpec((blk, D), lambda i: (i, 0)),
            pl.BlockSpec((blk, D), lambda i: (i, 0)),
            pl.BlockSpec((wrows, D), lambda i: (i, 0)),
            pl.BlockSpec((wrows, D), lambda i: (i, 0)),
            pl.BlockSpec((1, 1, blk), lambda i: (i, 0, 0)),
            pl.BlockSpec((G, DG), lambda i: (0, 0)),
            pl.BlockSpec((D, D), lambda i: (0, 0)),
            pl.BlockSpec((1, D), lambda i: (0, 0)),
            pl.BlockSpec((D, D), lambda i: (0, 0)),
            pl.BlockSpec((D, D), lambda i: (0, 0)),
            pl.BlockSpec((DG, D), lambda i: (0, 0)),
            pl.BlockSpec((1, D), lambda i: (0, 0)),
            pl.BlockSpec((D, D), lambda i: (0, 0)),
            pl.BlockSpec((1, D), lambda i: (0, 0)),
        ],
        out_specs=pl.BlockSpec((blk, D), lambda i: (i, 0)),
        out_shape=jax.ShapeDtypeStruct((NP, D), jnp.float32),
    )(xp, s_parts[0], s_parts[1], wc_parts[0], wc_parts[1],
      nb3, u, w2, b2.reshape(1, D),
      w3a, w3b, w3c, b3.reshape(1, D), w4, b4.reshape(1, D))


def kernel(x, edge_index, edge_attr, u, node_batch, wts,
           W1, b1, W2, b2, W3, b3, W4, b4):
    row = edge_index[0].astype(jnp.int32)
    col = edge_index[1].astype(jnp.int32)
    w = wts[:, 0]

    a = _proj_a(x, W1[:D])
    b = _proj_b(edge_attr, W1[D:], b1)

    meta = jnp.stack([row.reshape(NW, NCHUNK, CH),
                      col.reshape(NW, NCHUNK, CH),
                      (col // 64).reshape(NW, NCHUNK, CH)], axis=2)
    meta = jnp.concatenate(
        [meta, jnp.zeros((NW, 1, 3, CH), jnp.int32)], axis=1)
    wpad = jnp.concatenate(
        [w.reshape(NW, NCHUNK, CH), jnp.zeros((NW, 1, CH), jnp.float32)],
        axis=1)
    b4d = b.reshape(NW, NCHUNK, CH, H)
    zeros = jnp.zeros((NP, D), jnp.float32)

    s_parts, wc_parts = _edge_stage(meta, wpad, a, b4d, zeros)

    xp = jnp.concatenate([x, jnp.zeros((NP - N, D), jnp.float32)], axis=0)
    nbp = jnp.concatenate([node_batch,
                           jnp.zeros((NP - N,), jnp.int32)], axis=0)
    nb3 = nbp.reshape(NP // 2048, 1, 2048)
    out = _update_stage(xp, s_parts, wc_parts, nb3, u,
                        W2, b2, W3[:D], W3[D:2 * D], W3[2 * D:], b3, W4, b4)
    return out[:N]


# speedup vs baseline: 2.0738x; 1.2307x over previous
import jax
import jax.numpy as jnp
from jax import lax
from jax.experimental import pallas as pl
from jax.experimental.pallas import tpu as pltpu
from jax.experimental.pallas import tpu_sc as plsc

N = 10000
E = 320000
D = 128
DE = 16
DG = 64
G = 64
H = 128

NC = 2
NS = 16
NW = NC * NS
EPT = E // NW
CH = 80
NCHUNK = EPT // CH
NSB = CH // 16
NP = 10240
RPT = NP // NS
WCR = NP // 64


def _proj_a_body(x_ref, w_ref, o_ref):
    o_ref[...] = jnp.dot(x_ref[...], w_ref[...],
                         preferred_element_type=jnp.float32)


def _proj_b_body(eat_ref, w_ref, b_ref, o_ref):
    o_ref[...] = lax.dot_general(
        eat_ref[...], w_ref[...],
        dimension_numbers=(((0,), (0,)), ((), ())),
        preferred_element_type=jnp.float32) + b_ref[...]


def _proj_a(x, w1a):
    blk = 2000
    return pl.pallas_call(
        _proj_a_body,
        grid=(N // blk,),
        in_specs=[
            pl.BlockSpec((blk, D), lambda i: (i, 0)),
            pl.BlockSpec((D, H), lambda i: (0, 0)),
        ],
        out_specs=pl.BlockSpec((blk, H), lambda i: (i, 0)),
        out_shape=jax.ShapeDtypeStruct((N, H), jnp.float32),
    )(x, w1a)


def _proj_b(eat, w1b, b1):
    blk = 6400
    return pl.pallas_call(
        _proj_b_body,
        grid=(E // blk,),
        in_specs=[
            pl.BlockSpec((DE, blk), lambda i: (0, i)),
            pl.BlockSpec((DE, H), lambda i: (0, 0)),
            pl.BlockSpec((1, H), lambda i: (0, 0)),
        ],
        out_specs=pl.BlockSpec((blk, H), lambda i: (i, 0)),
        out_shape=jax.ShapeDtypeStruct((E, H), jnp.float32),
    )(eat, w1b, b1.reshape(1, H))


def _edge_body(meta_hbm, wts_hbm, a_hbm, b_hbm, zeros_hbm, out_hbm, outwc_hbm,
               s_acc, wc_acc, meta_v, wts_v, a_v, b_v, o_v, o2_v,
               sem_meta, sem_ab, sem_st):
    c = lax.axis_index("c")
    s = lax.axis_index("s")
    wid = s * NC + c
    pltpu.sync_copy(zeros_hbm.at[pl.ds(s * RPT, RPT)],
                    s_acc.at[pl.ds(s * RPT, RPT)])

    @pl.when(s < 5)
    def _zero_wc():
        pltpu.sync_copy(zeros_hbm.at[pl.ds(s * 32, 32)],
                        wc_acc.at[pl.ds(s * 32, 32)])

    plsc.subcore_barrier()

    iota16 = lax.broadcasted_iota(jnp.int32, (16,), 0)

    pltpu.async_copy(meta_hbm.at[wid, 0], meta_v.at[0], sem_meta)
    pltpu.async_copy(wts_hbm.at[wid, 0], wts_v.at[0], sem_meta)

    def chunk(i, carry):
        slot = lax.rem(i, 2)
        nslot = lax.rem(i + 1, 2)
        pltpu.make_async_copy(meta_hbm.at[wid, i], meta_v.at[slot],
                              sem_meta).wait()
        pltpu.make_async_copy(wts_hbm.at[wid, i], wts_v.at[slot],
                              sem_meta).wait()
        ga = pltpu.async_copy(a_hbm.at[meta_v.at[slot, 0]], a_v, sem_ab)
        gb = pltpu.async_copy(b_hbm.at[wid, i], b_v, sem_ab)

        @pl.when(i > 0)
        def _drain():
            pltpu.make_async_copy(o_v, s_acc.at[meta_v.at[slot, 1]],
                                  sem_st).wait()
            pltpu.make_async_copy(o2_v, wc_acc.at[meta_v.at[slot, 2]],
                                  sem_st).wait()

        pltpu.async_copy(meta_hbm.at[wid, i + 1], meta_v.at[nslot], sem_meta)
        pltpu.async_copy(wts_hbm.at[wid, i + 1], wts_v.at[nslot], sem_meta)
        ga.wait()
        gb.wait()

        def subblock(sb, carry2):
            wv = wts_v[slot, pl.ds(sb * 16, 16)]
            cv = meta_v[slot, 1, pl.ds(sb * 16, 16)]
            l2 = 2 * (cv % 64)
            for ee in range(16):
                e = sb * 16 + ee
                w = wv[ee]
                t = l2[ee]
                for j in range(D // 16):
                    lo = iota16 + (16 * j)
                    o2_v[e, pl.ds(16 * j, 16)] = jnp.where(
                        lo == t, w, jnp.where(lo == t + 1, 1.0, 0.0))
                for j in range(D // 16):
                    sl = pl.ds(16 * j, 16)
                    o_v[e, sl] = jnp.maximum(a_v[e, sl] + b_v[e, sl],
                                             0.0) * w
            return carry2

        lax.fori_loop(0, NSB, subblock, 0, unroll=False)
        pltpu.async_copy(o_v, s_acc.at[meta_v.at[slot, 1]], sem_st,
                         add=True)
        pltpu.async_copy(o2_v, wc_acc.at[meta_v.at[slot, 2]], sem_st,
                         add=True)
        return carry

    lax.fori_loop(0, NCHUNK, chunk, 0, unroll=False)
    lslot = lax.rem(NCHUNK - 1, 2)
    pltpu.make_async_copy(o_v, s_acc.at[meta_v.at[lslot, 1]], sem_st).wait()
    pltpu.make_async_copy(o2_v, wc_acc.at[meta_v.at[lslot, 2]],
                          sem_st).wait()
    pltpu.make_async_copy(meta_hbm.at[wid, NCHUNK],
                          meta_v.at[lax.rem(NCHUNK, 2)], sem_meta).wait()
    pltpu.make_async_copy(wts_hbm.at[wid, NCHUNK],
                          wts_v.at[lax.rem(NCHUNK, 2)], sem_meta).wait()
    plsc.subcore_barrier()
    pltpu.sync_copy(s_acc.at[pl.ds(s * RPT, RPT)],
                    out_hbm.at[c, pl.ds(s * RPT, RPT)])

    @pl.when(s == 0)
    def _dump_wc():
        pltpu.sync_copy(wc_acc, outwc_hbm.at[c])


def _edge_stage(meta, wts, a, b, zeros):
    mesh = plsc.VectorSubcoreMesh(core_axis_name="c", subcore_axis_name="s",
                                  num_cores=NC, num_subcores=NS)
    fn = pl.kernel(
        _edge_body,
        out_type=(jax.ShapeDtypeStruct((NC, NP, D), jnp.float32),
                  jax.ShapeDtypeStruct((NC, WCR, D), jnp.float32)),
        mesh=mesh,
        scratch_types=[
            pltpu.VMEM_SHARED((NP, D), jnp.float32),
            pltpu.VMEM_SHARED((WCR, D), jnp.float32),
            pltpu.VMEM((2, 3, CH), jnp.int32),
            pltpu.VMEM((2, CH), jnp.float32),
            pltpu.VMEM((CH, D), jnp.float32),
            pltpu.VMEM((CH, D), jnp.float32),
            pltpu.VMEM((CH, D), jnp.float32),
            pltpu.VMEM((CH, D), jnp.float32),
            pltpu.SemaphoreType.DMA,
            pltpu.SemaphoreType.DMA,
            pltpu.SemaphoreType.DMA,
        ],
        compiler_params=pltpu.CompilerParams(use_tc_tiling_on_sc=False),
    )
    return fn(meta, wts, a, b, zeros)


def _update_body(x_ref, s0_ref, s1_ref, wc0_ref, wc1_ref,
                 nb_ref, u_ref,
                 w2_ref, b2_ref, w3a_ref, w3b_ref, w3c_ref, b3_ref,
                 w4_ref, b4_ref, o_ref):
    blk = x_ref.shape[0]
    wrows = blk // 64
    msg = s0_ref[...] + s1_ref[...]
    wcp = wc0_ref[...] + wc1_ref[...]
    rep_oh = (lax.broadcasted_iota(jnp.int32, (blk, wrows), 0) // 64
              == lax.broadcasted_iota(jnp.int32, (blk, wrows), 1)
              ).astype(jnp.float32)
    rep = jnp.dot(rep_oh, wcp, preferred_element_type=jnp.float32)
    lane2 = 2 * (lax.broadcasted_iota(jnp.int32, (blk, D), 0) % 64)
    lid = lax.broadcasted_iota(jnp.int32, (blk, D), 1)
    wsum = jnp.sum(jnp.where(lid == lane2, rep, 0.0), axis=1, keepdims=True)
    cnt = jnp.sum(jnp.where(lid == lane2 + 1, rep, 0.0),
                  axis=1, keepdims=True)
    rc = 1.0 / jnp.maximum(cnt, 1.0)
    recv = (jnp.dot(msg, w2_ref[...], preferred_element_type=jnp.float32)
            + wsum * b2_ref[...]) * rc
    nb = nb_ref[0, 0, :]
    onehot = (nb[:, None] == lax.broadcasted_iota(jnp.int32, (blk, G), 1)
              ).astype(jnp.float32)
    uproj = jnp.dot(u_ref[...], w3c_ref[...],
                    preferred_element_type=jnp.float32)
    pre = (jnp.dot(x_ref[...], w3a_ref[...],
                   preferred_element_type=jnp.float32)
           + jnp.dot(recv, w3b_ref[...], preferred_element_type=jnp.float32)
           + jnp.dot(onehot, uproj, preferred_element_type=jnp.float32)
           + b3_ref[...])
    h2 = jnp.maximum(pre, 0.0)
    o_ref[...] = (jnp.dot(h2, w4_ref[...], preferred_element_type=jnp.float32)
                  + b4_ref[...])


def _update_stage(xp, s_parts, wc_parts, nb3, u,
                  w2, b2, w3a, w3b, w3c, b3, w4, b4):
    blk = 2048
    grid = NP // blk
    wrows = blk // 64
    return pl.pallas_call(
        _update_body,
        grid=(grid,),
        in_specs=[
            pl.BlockSpec((blk, D), lambda i: (i, 0)),
            pl.BlockSpec((blk, D), lambda i: (i, 0)),
            pl.BlockSpec((blk, D), lambda i: (i, 0)),
            pl.BlockSpec((wrows, D), lambda i: (i, 0)),
            pl.BlockSpec((wrows, D), lambda i: (i, 0)),
            pl.BlockSpec((1, 1, blk), lambda i: (i, 0, 0)),
            pl.BlockSpec((G, DG), lambda i: (0, 0)),
            pl.BlockSpec((D, D), lambda i: (0, 0)),
            pl.BlockSpec((1, D), lambda i: (0, 0)),
            pl.BlockSpec((D, D), lambda i: (0, 0)),
            pl.BlockSpec((D, D), lambda i: (0, 0)),
            pl.BlockSpec((DG, D), lambda i: (0, 0)),
            pl.BlockSpec((1, D), lambda i: (0, 0)),
            pl.BlockSpec((D, D), lambda i: (0, 0)),
            pl.BlockSpec((1, D), lambda i: (0, 0)),
        ],
        out_specs=pl.BlockSpec((blk, D), lambda i: (i, 0)),
        out_shape=jax.ShapeDtypeStruct((NP, D), jnp.float32),
    )(xp, s_parts[0], s_parts[1], wc_parts[0], wc_parts[1],
      nb3, u, w2, b2.reshape(1, D),
      w3a, w3b, w3c, b3.reshape(1, D), w4, b4.reshape(1, D))


def kernel(x, edge_index, edge_attr, u, node_batch, wts,
           W1, b1, W2, b2, W3, b3, W4, b4):
    row = edge_index[0].astype(jnp.int32)
    col = edge_index[1].astype(jnp.int32)
    w = wts[:, 0]

    a = _proj_a(x, W1[:D])
    b = _proj_b(edge_attr.T, W1[D:], b1)

    meta = jnp.stack([row.reshape(NW, NCHUNK, CH),
                      col.reshape(NW, NCHUNK, CH),
                      jnp.right_shift(col, 6).reshape(NW, NCHUNK, CH)],
                     axis=2)
    meta = jnp.concatenate(
        [meta, jnp.zeros((NW, 1, 3, CH), jnp.int32)], axis=1)
    wpad = jnp.concatenate(
        [w.reshape(NW, NCHUNK, CH), jnp.zeros((NW, 1, CH), jnp.float32)],
        axis=1)
    b4d = b.reshape(NW, NCHUNK, CH, H)
    zeros = jnp.zeros((NP, D), jnp.float32)

    s_parts, wc_parts = _edge_stage(meta, wpad, a, b4d, zeros)

    nbp = jnp.concatenate([node_batch,
                           jnp.zeros((NP - N,), jnp.int32)], axis=0)
    nb3 = nbp.reshape(NP // 2048, 1, 2048)
    out = _update_stage(x, s_parts, wc_parts, nb3, u,
                        W2, b2, W3[:D], W3[D:2 * D], W3[2 * D:], b3, W4, b4)
    return out[:N]


# speedup vs baseline: 2.2421x; 1.0811x over previous
import jax
import jax.numpy as jnp
from jax import lax
from jax.experimental import pallas as pl
from jax.experimental.pallas import tpu as pltpu
from jax.experimental.pallas import tpu_sc as plsc

N = 10000
E = 320000
D = 128
DE = 16
DG = 64
G = 64
H = 128

NC = 2
NS = 16
NW = NC * NS
EPT = E // NW
CH = 80
NCHUNK = EPT // CH
NSB = CH // 16
NP = 10240
RPT = NP // NS
WCR = NP // 64


def _proj_a_body(x_ref, w_ref, o_ref):
    o_ref[...] = jnp.dot(x_ref[...], w_ref[...],
                         preferred_element_type=jnp.float32)


def _proj_b_body(eat_ref, w_ref, b_ref, o_ref):
    o_ref[...] = lax.dot_general(
        eat_ref[...], w_ref[...],
        dimension_numbers=(((0,), (0,)), ((), ())),
        preferred_element_type=jnp.float32) + b_ref[...]


def _proj_a(x, w1a):
    blk = 2000
    return pl.pallas_call(
        _proj_a_body,
        grid=(N // blk,),
        in_specs=[
            pl.BlockSpec((blk, D), lambda i: (i, 0)),
            pl.BlockSpec((D, H), lambda i: (0, 0)),
        ],
        out_specs=pl.BlockSpec((blk, H), lambda i: (i, 0)),
        out_shape=jax.ShapeDtypeStruct((N, H), jnp.float32),
    )(x, w1a)


def _proj_b(eat, w1b, b1):
    blk = 6400
    return pl.pallas_call(
        _proj_b_body,
        grid=(E // blk,),
        in_specs=[
            pl.BlockSpec((DE, blk), lambda i: (0, i)),
            pl.BlockSpec((DE, H), lambda i: (0, 0)),
            pl.BlockSpec((1, H), lambda i: (0, 0)),
        ],
        out_specs=pl.BlockSpec((blk, H), lambda i: (i, 0)),
        out_shape=jax.ShapeDtypeStruct((E, H), jnp.float32),
    )(eat, w1b, b1.reshape(1, H))


def _edge_body(rc_hbm, wts_hbm, a_hbm, b_hbm, zeros_hbm, out_hbm, outwc_hbm,
               s_acc, wc_acc, rc_v, wts_v, wcidx_v, a_v, b_v, o_v, o2_v,
               sem_meta, sem_ab, sem_st):
    c = lax.axis_index("c")
    s = lax.axis_index("s")
    wid = s * NC + c
    pltpu.sync_copy(zeros_hbm.at[pl.ds(s * RPT, RPT)],
                    s_acc.at[pl.ds(s * RPT, RPT)])

    @pl.when(s < 5)
    def _zero_wc():
        pltpu.sync_copy(zeros_hbm.at[pl.ds(s * 32, 32)],
                        wc_acc.at[pl.ds(s * 32, 32)])

    plsc.subcore_barrier()

    iota16 = lax.broadcasted_iota(jnp.int32, (16,), 0)

    pltpu.async_copy(rc_hbm.at[pl.ds(0, 2), wid, 0], rc_v.at[0], sem_meta)
    pltpu.async_copy(wts_hbm.at[wid, 0], wts_v.at[0], sem_meta)

    def chunk(i, carry):
        slot = lax.rem(i, 2)
        nslot = lax.rem(i + 1, 2)
        pltpu.make_async_copy(rc_hbm.at[pl.ds(0, 2), wid, i], rc_v.at[slot],
                              sem_meta).wait()
        pltpu.make_async_copy(wts_hbm.at[wid, i], wts_v.at[slot],
                              sem_meta).wait()
        ga = pltpu.async_copy(a_hbm.at[rc_v.at[slot, 0]], a_v, sem_ab)
        gb = pltpu.async_copy(b_hbm.at[wid, i], b_v, sem_ab)

        @pl.when(i > 0)
        def _drain():
            pltpu.make_async_copy(o_v, s_acc.at[rc_v.at[slot, 1]],
                                  sem_st).wait()
            pltpu.make_async_copy(o2_v, wc_acc.at[wcidx_v],
                                  sem_st).wait()

        pltpu.async_copy(rc_hbm.at[pl.ds(0, 2), wid, i + 1], rc_v.at[nslot],
                         sem_meta)
        pltpu.async_copy(wts_hbm.at[wid, i + 1], wts_v.at[nslot], sem_meta)
        ga.wait()
        gb.wait()

        def subblock(sb, carry2):
            wv = wts_v[slot, pl.ds(sb * 16, 16)]
            cv = rc_v[slot, 1, pl.ds(sb * 16, 16)]
            wcidx_v[pl.ds(sb * 16, 16)] = jnp.right_shift(cv, 6)
            l2 = 2 * (cv % 64)
            for ee in range(16):
                e = sb * 16 + ee
                w = wv[ee]
                t = l2[ee]
                for j in range(D // 16):
                    lo = iota16 + (16 * j)
                    o2_v[e, pl.ds(16 * j, 16)] = jnp.where(
                        lo == t, w, jnp.where(lo == t + 1, 1.0, 0.0))
                for j in range(D // 16):
                    sl = pl.ds(16 * j, 16)
                    o_v[e, sl] = jnp.maximum(a_v[e, sl] + b_v[e, sl],
                                             0.0) * w
            return carry2

        lax.fori_loop(0, NSB, subblock, 0, unroll=False)
        pltpu.async_copy(o_v, s_acc.at[rc_v.at[slot, 1]], sem_st,
                         add=True)
        pltpu.async_copy(o2_v, wc_acc.at[wcidx_v], sem_st, add=True)
        return carry

    lax.fori_loop(0, NCHUNK, chunk, 0, unroll=False)
    lslot = lax.rem(NCHUNK - 1, 2)
    pltpu.make_async_copy(o_v, s_acc.at[rc_v.at[lslot, 1]], sem_st).wait()
    pltpu.make_async_copy(o2_v, wc_acc.at[wcidx_v], sem_st).wait()
    pltpu.make_async_copy(rc_hbm.at[pl.ds(0, 2), wid, NCHUNK],
                          rc_v.at[lax.rem(NCHUNK, 2)], sem_meta).wait()
    pltpu.make_async_copy(wts_hbm.at[wid, NCHUNK],
                          wts_v.at[lax.rem(NCHUNK, 2)], sem_meta).wait()
    plsc.subcore_barrier()
    pltpu.sync_copy(s_acc.at[pl.ds(s * RPT, RPT)],
                    out_hbm.at[c, pl.ds(s * RPT, RPT)])

    @pl.when(s == 0)
    def _dump_wc():
        pltpu.sync_copy(wc_acc, outwc_hbm.at[c])


def _edge_stage(rc, wts, a, b, zeros):
    mesh = plsc.VectorSubcoreMesh(core_axis_name="c", subcore_axis_name="s",
                                  num_cores=NC, num_subcores=NS)
    fn = pl.kernel(
        _edge_body,
        out_type=(jax.ShapeDtypeStruct((NC, NP, D), jnp.float32),
                  jax.ShapeDtypeStruct((NC, WCR, D), jnp.float32)),
        mesh=mesh,
        scratch_types=[
            pltpu.VMEM_SHARED((NP, D), jnp.float32),
            pltpu.VMEM_SHARED((WCR, D), jnp.float32),
            pltpu.VMEM((2, 2, CH), jnp.int32),
            pltpu.VMEM((2, CH), jnp.float32),
            pltpu.VMEM((CH,), jnp.int32),
            pltpu.VMEM((CH, D), jnp.float32),
            pltpu.VMEM((CH, D), jnp.float32),
            pltpu.VMEM((CH, D), jnp.float32),
            pltpu.VMEM((CH, D), jnp.float32),
            pltpu.SemaphoreType.DMA,
            pltpu.SemaphoreType.DMA,
            pltpu.SemaphoreType.DMA,
        ],
        compiler_params=pltpu.CompilerParams(use_tc_tiling_on_sc=False),
    )
    return fn(rc, wts, a, b, zeros)


def _update_body(x_ref, s0_ref, s1_ref, wc0_ref, wc1_ref,
                 nb_ref, u_ref,
                 w2_ref, b2_ref, w3a_ref, w3b_ref, w3c_ref, b3_ref,
                 w4_ref, b4_ref, o_ref):
    blk = x_ref.shape[0]
    wrows = blk // 64
    msg = s0_ref[...] + s1_ref[...]
    wcp = wc0_ref[...] + wc1_ref[...]
    rep_oh = (lax.broadcasted_iota(jnp.int32, (blk, wrows), 0) // 64
              == lax.broadcasted_iota(jnp.int32, (blk, wrows), 1)
              ).astype(jnp.float32)
    rep = jnp.dot(rep_oh, wcp, preferred_element_type=jnp.float32)
    lane2 = 2 * (lax.broadcasted_iota(jnp.int32, (blk, D), 0) % 64)
    lid = lax.broadcasted_iota(jnp.int32, (blk, D), 1)
    wsum = jnp.sum(jnp.where(lid == lane2, rep, 0.0), axis=1, keepdims=True)
    cnt = jnp.sum(jnp.where(lid == lane2 + 1, rep, 0.0),
                  axis=1, keepdims=True)
    rc = 1.0 / jnp.maximum(cnt, 1.0)
    recv = (jnp.dot(msg, w2_ref[...], preferred_element_type=jnp.float32)
            + wsum * b2_ref[...]) * rc
    nb = nb_ref[0, 0, :]
    onehot = (nb[:, None] == lax.broadcasted_iota(jnp.int32, (blk, G), 1)
              ).astype(jnp.float32)
    uproj = jnp.dot(u_ref[...], w3c_ref[...],
                    preferred_element_type=jnp.float32)
    pre = (jnp.dot(x_ref[...], w3a_ref[...],
                   preferred_element_type=jnp.float32)
           + jnp.dot(recv, w3b_ref[...], preferred_element_type=jnp.float32)
           + jnp.dot(onehot, uproj, preferred_element_type=jnp.float32)
           + b3_ref[...])
    h2 = jnp.maximum(pre, 0.0)
    o_ref[...] = (jnp.dot(h2, w4_ref[...], preferred_element_type=jnp.float32)
                  + b4_ref[...])


def _update_stage(xp, s_parts, wc_parts, nb3, u,
                  w2, b2, w3a, w3b, w3c, b3, w4, b4):
    blk = 2048
    grid = NP // blk
    wrows = blk // 64
    return pl.pallas_call(
        _update_body,
        grid=(grid,),
        in_specs=[
            pl.BlockSpec((blk, D), lambda i: (i, 0)),
            pl.BlockSpec((blk, D), lambda i: (i, 0)),
            pl.BlockSpec((blk, D), lambda i: (i, 0)),
            pl.BlockSpec((wrows, D), lambda i: (i, 0)),
            pl.BlockSpec((wrows, D), lambda i: (i, 0)),
            pl.BlockSpec((1, 1, blk), lambda i: (i, 0, 0)),
            pl.BlockSpec((G, DG), lambda i: (0, 0)),
            pl.BlockSpec((D, D), lambda i: (0, 0)),
            pl.BlockSpec((1, D), lambda i: (0, 0)),
            pl.BlockSpec((D, D), lambda i: (0, 0)),
            pl.BlockSpec((D, D), lambda i: (0, 0)),
            pl.BlockSpec((DG, D), lambda i: (0, 0)),
            pl.BlockSpec((1, D), lambda i: (0, 0)),
            pl.BlockSpec((D, D), lambda i: (0, 0)),
            pl.BlockSpec((1, D), lambda i: (0, 0)),
        ],
        out_specs=pl.BlockSpec((blk, D), lambda i: (i, 0)),
        out_shape=jax.ShapeDtypeStruct((NP, D), jnp.float32),
    )(xp, s_parts[0], s_parts[1], wc_parts[0], wc_parts[1],
      nb3, u, w2, b2.reshape(1, D),
      w3a, w3b, w3c, b3.reshape(1, D), w4, b4.reshape(1, D))


def kernel(x, edge_index, edge_attr, u, node_batch, wts,
           W1, b1, W2, b2, W3, b3, W4, b4):
    a = _proj_a(x, W1[:D])
    b = _proj_b(edge_attr.T, W1[D:], b1)

    rc = jnp.concatenate(
        [edge_index.reshape(2, NW, NCHUNK, CH),
         jnp.zeros((2, NW, 1, CH), jnp.int32)], axis=2)
    wpad = jnp.concatenate(
        [wts.reshape(NW, NCHUNK, CH),
         jnp.zeros((NW, 1, CH), jnp.float32)], axis=1)
    b4d = b.reshape(NW, NCHUNK, CH, H)
    zeros = jnp.zeros((NP, D), jnp.float32)

    s_parts, wc_parts = _edge_stage(rc, wpad, a, b4d, zeros)

    nbp = jnp.concatenate([node_batch,
                           jnp.zeros((NP - N,), jnp.int32)], axis=0)
    nb3 = nbp.reshape(NP // 2048, 1, 2048)
    out = _update_stage(x, s_parts, wc_parts, nb3, u,
                        W2, b2, W3[:D], W3[D:2 * D], W3[2 * D:], b3, W4, b4)
    return out[:N]


# speedup vs baseline: 2.2789x; 1.0164x over previous
import jax
import jax.numpy as jnp
from jax import lax
from jax.experimental import pallas as pl
from jax.experimental.pallas import tpu as pltpu
from jax.experimental.pallas import tpu_sc as plsc

N = 10000
E = 320000
D = 128
DE = 16
DG = 64
G = 64
H = 128

NC = 2
NS = 16
NW = NC * NS
EPT = E // NW
CH = 80
NCHUNK = EPT // CH
NSB = CH // 16
NP = 10240
RPT = NP // NS
WCR = NP // 64


def _proj_a_body(x_ref, w_ref, o_ref):
    o_ref[...] = jnp.dot(x_ref[...], w_ref[...],
                         preferred_element_type=jnp.float32)


def _proj_b_body(eat_ref, w_ref, b_ref, o_ref):
    o_ref[...] = lax.dot_general(
        eat_ref[...], w_ref[...],
        dimension_numbers=(((0,), (0,)), ((), ())),
        preferred_element_type=jnp.float32) + b_ref[...]


def _proj_a(x, w1a):
    blk = 2000
    return pl.pallas_call(
        _proj_a_body,
        grid=(N // blk,),
        in_specs=[
            pl.BlockSpec((blk, D), lambda i: (i, 0)),
            pl.BlockSpec((D, H), lambda i: (0, 0)),
        ],
        out_specs=pl.BlockSpec((blk, H), lambda i: (i, 0)),
        out_shape=jax.ShapeDtypeStruct((N, H), jnp.float32),
    )(x, w1a)


def _proj_b(eat, w1b, b1):
    blk = 6400
    return pl.pallas_call(
        _proj_b_body,
        grid=(E // blk,),
        in_specs=[
            pl.BlockSpec((DE, blk), lambda i: (0, i)),
            pl.BlockSpec((DE, H), lambda i: (0, 0)),
            pl.BlockSpec((1, H), lambda i: (0, 0)),
        ],
        out_specs=pl.BlockSpec((blk, H), lambda i: (i, 0)),
        out_shape=jax.ShapeDtypeStruct((E, H), jnp.float32),
    )(eat, w1b, b1.reshape(1, H))


def _edge_body(rc_hbm, wts_hbm, a_hbm, b_hbm, zeros_hbm, out_hbm, outwc_hbm,
               s_acc, wc_acc, rc_v, wts_v, wcidx_v, a_v, b_v, o_v, o2_v,
               sem_meta, sem_ab, sem_st):
    c = lax.axis_index("c")
    s = lax.axis_index("s")
    wid = s * NC + c
    pltpu.sync_copy(zeros_hbm.at[pl.ds(s * RPT, RPT)],
                    s_acc.at[pl.ds(s * RPT, RPT)])

    @pl.when(s < 5)
    def _zero_wc():
        pltpu.sync_copy(zeros_hbm.at[pl.ds(s * 32, 32)],
                        wc_acc.at[pl.ds(s * 32, 32)])

    plsc.subcore_barrier()

    iota16 = lax.broadcasted_iota(jnp.int32, (16,), 0)

    pltpu.async_copy(rc_hbm.at[pl.ds(0, 2), wid, 0], rc_v.at[0], sem_meta)
    pltpu.async_copy(wts_hbm.at[wid, 0], wts_v.at[0], sem_meta)

    def chunk(i, carry):
        slot = lax.rem(i, 2)
        nslot = lax.rem(i + 1, 2)
        pltpu.make_async_copy(rc_hbm.at[pl.ds(0, 2), wid, i], rc_v.at[slot],
                              sem_meta).wait()
        pltpu.make_async_copy(wts_hbm.at[wid, i], wts_v.at[slot],
                              sem_meta).wait()
        ga = pltpu.async_copy(a_hbm.at[rc_v.at[slot, 0]], a_v, sem_ab)
        gb = pltpu.async_copy(b_hbm.at[wid, i], b_v, sem_ab)

        @pl.when(i > 0)
        def _drain():
            pltpu.make_async_copy(o_v, s_acc.at[rc_v.at[slot, 1]],
                                  sem_st).wait()
            pltpu.make_async_copy(o2_v, wc_acc.at[wcidx_v],
                                  sem_st).wait()

        pltpu.async_copy(rc_hbm.at[pl.ds(0, 2), wid, i + 1], rc_v.at[nslot],
                         sem_meta)
        pltpu.async_copy(wts_hbm.at[wid, i + 1], wts_v.at[nslot], sem_meta)
        ga.wait()
        gb.wait()

        def subblock(sb, carry2):
            wv = wts_v[slot, pl.ds(sb * 16, 16)]
            cv = rc_v[slot, 1, pl.ds(sb * 16, 16)]
            wcidx_v[pl.ds(sb * 16, 16)] = jnp.right_shift(cv, 6)
            l2 = 2 * (cv % 64)
            for ee in range(16):
                e = sb * 16 + ee
                w = wv[ee]
                t = l2[ee]
                for j in range(D // 16):
                    lo = iota16 + (16 * j)
                    o2_v[e, pl.ds(16 * j, 16)] = jnp.where(
                        lo == t, w, jnp.where(lo == t + 1, 1.0, 0.0))
                for j in range(D // 16):
                    sl = pl.ds(16 * j, 16)
                    o_v[e, sl] = jnp.maximum(a_v[e, sl] + b_v[e, sl],
                                             0.0) * w
            return carry2

        lax.fori_loop(0, NSB, subblock, 0, unroll=False)
        pltpu.async_copy(o_v, s_acc.at[rc_v.at[slot, 1]], sem_st,
                         add=True)
        pltpu.async_copy(o2_v, wc_acc.at[wcidx_v], sem_st, add=True)
        return carry

    lax.fori_loop(0, NCHUNK, chunk, 0, unroll=False)
    lslot = lax.rem(NCHUNK - 1, 2)
    pltpu.make_async_copy(o_v, s_acc.at[rc_v.at[lslot, 1]], sem_st).wait()
    pltpu.make_async_copy(o2_v, wc_acc.at[wcidx_v], sem_st).wait()
    pltpu.make_async_copy(rc_hbm.at[pl.ds(0, 2), wid, NCHUNK],
                          rc_v.at[lax.rem(NCHUNK, 2)], sem_meta).wait()
    pltpu.make_async_copy(wts_hbm.at[wid, NCHUNK],
                          wts_v.at[lax.rem(NCHUNK, 2)], sem_meta).wait()
    plsc.subcore_barrier()
    pltpu.sync_copy(s_acc.at[pl.ds(s * RPT, RPT)],
                    out_hbm.at[c, pl.ds(s * RPT, RPT)])

    @pl.when(s == 0)
    def _dump_wc():
        pltpu.sync_copy(wc_acc, outwc_hbm.at[c])


def _edge_stage(rc, wts, a, b, zeros):
    mesh = plsc.VectorSubcoreMesh(core_axis_name="c", subcore_axis_name="s",
                                  num_cores=NC, num_subcores=NS)
    fn = pl.kernel(
        _edge_body,
        out_type=(jax.ShapeDtypeStruct((NC, NP, D), jnp.float32),
                  jax.ShapeDtypeStruct((NC, WCR, D), jnp.float32)),
        mesh=mesh,
        scratch_types=[
            pltpu.VMEM_SHARED((NP, D), jnp.float32),
            pltpu.VMEM_SHARED((WCR, D), jnp.float32),
            pltpu.VMEM((2, 2, CH), jnp.int32),
            pltpu.VMEM((2, CH), jnp.float32),
            pltpu.VMEM((CH,), jnp.int32),
            pltpu.VMEM((CH, D), jnp.float32),
            pltpu.VMEM((CH, D), jnp.float32),
            pltpu.VMEM((CH, D), jnp.float32),
            pltpu.VMEM((CH, D), jnp.float32),
            pltpu.SemaphoreType.DMA,
            pltpu.SemaphoreType.DMA,
            pltpu.SemaphoreType.DMA,
        ],
        compiler_params=pltpu.CompilerParams(use_tc_tiling_on_sc=False),
    )
    return fn(rc, wts, a, b, zeros)


def _update_body(x_ref, s0_ref, s1_ref, wc0_ref, wc1_ref,
                 nb_ref, u_ref,
                 w2_ref, b2_ref, w3a_ref, w3b_ref, w3c_ref, b3_ref,
                 w4_ref, b4_ref, o_ref):
    blk = x_ref.shape[0]
    wrows = blk // 64
    msg = s0_ref[0] + s1_ref[0]
    wcp = wc0_ref[0] + wc1_ref[0]
    rep_oh = (lax.broadcasted_iota(jnp.int32, (blk, wrows), 0) // 64
              == lax.broadcasted_iota(jnp.int32, (blk, wrows), 1)
              ).astype(jnp.float32)
    rep = jnp.dot(rep_oh, wcp, preferred_element_type=jnp.float32)
    lane2 = 2 * (lax.broadcasted_iota(jnp.int32, (blk, D), 0) % 64)
    lid = lax.broadcasted_iota(jnp.int32, (blk, D), 1)
    wsum = jnp.sum(jnp.where(lid == lane2, rep, 0.0), axis=1, keepdims=True)
    cnt = jnp.sum(jnp.where(lid == lane2 + 1, rep, 0.0),
                  axis=1, keepdims=True)
    rc = 1.0 / jnp.maximum(cnt, 1.0)
    recv = (jnp.dot(msg, w2_ref[...], preferred_element_type=jnp.float32)
            + wsum * b2_ref[...]) * rc
    nb = nb_ref[0, 0, :]
    onehot = (nb[:, None] == lax.broadcasted_iota(jnp.int32, (blk, G), 1)
              ).astype(jnp.float32)
    uproj = jnp.dot(u_ref[...], w3c_ref[...],
                    preferred_element_type=jnp.float32)
    pre = (jnp.dot(x_ref[...], w3a_ref[...],
                   preferred_element_type=jnp.float32)
           + jnp.dot(recv, w3b_ref[...], preferred_element_type=jnp.float32)
           + jnp.dot(onehot, uproj, preferred_element_type=jnp.float32)
           + b3_ref[...])
    h2 = jnp.maximum(pre, 0.0)
    o_ref[...] = (jnp.dot(h2, w4_ref[...], preferred_element_type=jnp.float32)
                  + b4_ref[...])


def _update_stage(xp, s_parts, wc_parts, nb3, u,
                  w2, b2, w3a, w3b, w3c, b3, w4, b4):
    blk = 2048
    grid = NP // blk
    wrows = blk // 64
    return pl.pallas_call(
        _update_body,
        grid=(grid,),
        in_specs=[
            pl.BlockSpec((blk, D), lambda i: (i, 0)),
            pl.BlockSpec((1, blk, D), lambda i: (0, i, 0)),
            pl.BlockSpec((1, blk, D), lambda i: (1, i, 0)),
            pl.BlockSpec((1, wrows, D), lambda i: (0, i, 0)),
            pl.BlockSpec((1, wrows, D), lambda i: (1, i, 0)),
            pl.BlockSpec((1, 1, blk), lambda i: (i, 0, 0)),
            pl.BlockSpec((G, DG), lambda i: (0, 0)),
            pl.BlockSpec((D, D), lambda i: (0, 0)),
            pl.BlockSpec((1, D), lambda i: (0, 0)),
            pl.BlockSpec((D, D), lambda i: (0, 0)),
            pl.BlockSpec((D, D), lambda i: (0, 0)),
            pl.BlockSpec((DG, D), lambda i: (0, 0)),
            pl.BlockSpec((1, D), lambda i: (0, 0)),
            pl.BlockSpec((D, D), lambda i: (0, 0)),
            pl.BlockSpec((1, D), lambda i: (0, 0)),
        ],
        out_specs=pl.BlockSpec((blk, D), lambda i: (i, 0)),
        out_shape=jax.ShapeDtypeStruct((NP, D), jnp.float32),
    )(xp, s_parts, s_parts, wc_parts, wc_parts,
      nb3, u, w2, b2.reshape(1, D),
      w3a, w3b, w3c, b3.reshape(1, D), w4, b4.reshape(1, D))


def kernel(x, edge_index, edge_attr, u, node_batch, wts,
           W1, b1, W2, b2, W3, b3, W4, b4):
    a = _proj_a(x, W1[:D])
    b = _proj_b(edge_attr.T, W1[D:], b1)

    rc = jnp.concatenate(
        [edge_index.reshape(2, NW, NCHUNK, CH),
         jnp.zeros((2, NW, 1, CH), jnp.int32)], axis=2)
    wpad = jnp.concatenate(
        [wts.reshape(NW, NCHUNK, CH),
         jnp.zeros((NW, 1, CH), jnp.float32)], axis=1)
    b4d = b.reshape(NW, NCHUNK, CH, H)
    zeros = jnp.zeros((NP, D), jnp.float32)

    s_parts, wc_parts = _edge_stage(rc, wpad, a, b4d, zeros)

    nbp = jnp.concatenate([node_batch,
                           jnp.zeros((NP - N,), jnp.int32)], axis=0)
    nb3 = nbp.reshape(NP // 2048, 1, 2048)
    out = _update_stage(x, s_parts, wc_parts, nb3, u,
                        W2, b2, W3[:D], W3[D:2 * D], W3[2 * D:], b3, W4, b4)
    return out[:N]
